# baseline (speedup 1.0000x reference)
import jax
import jax.numpy as jnp
from jax.experimental import pallas as pl

NUM_OUTPUT = 12
H = 100
AG = 130
VOICES = 5
GP = 128
B_BLK = 1024


def _poly_body(x_ref, wx_ref, wsb_ref, bias_ref, wlin_ref, blin_ref, g_ref,
               sampled_ref, probs_ref):
    x = x_ref[...]
    base = jnp.dot(x, wx_ref[...],
                   preferred_element_type=jnp.float32) + bias_ref[...]
    blk = x.shape[0]
    sampled = jnp.zeros((blk, NUM_OUTPUT), jnp.float32)
    banned = jnp.zeros((blk, NUM_OUTPUT), jnp.float32)
    sample_p = jnp.zeros((blk, NUM_OUTPUT), jnp.float32)
    ban_p = jnp.zeros((blk, NUM_OUTPUT), jnp.float32)
    col = jax.lax.broadcasted_iota(jnp.int32, (blk, 2 * NUM_OUTPUT), 1)
    for v in range(VOICES):
        sb = jnp.concatenate([sampled, banned], axis=1)
        gates = base + jnp.dot(sb, wsb_ref[...],
                               preferred_element_type=jnp.float32)
        i = jax.nn.sigmoid(gates[:, 0:H])
        g = jnp.tanh(gates[:, GP:GP + H])
        o = jax.nn.sigmoid(gates[:, 2 * GP:2 * GP + H])
        h = o * jnp.tanh(i * g)
        out = jnp.dot(h, wlin_ref[...],
                      preferred_element_type=jnp.float32) + blin_ref[...]
        cm = (1.0 - sampled) * (1.0 - banned)
        coeff = jnp.concatenate([cm, cm], axis=1)
        p = coeff * jnp.exp(out)
        p = p / jnp.sum(p, axis=1, keepdims=True)
        logits = jnp.where(p > 0, jnp.log(jnp.maximum(p, 1e-30)), -1e9)
        z = logits + g_ref[v]
        m = jnp.max(z, axis=1, keepdims=True)
        idx = jnp.min(jnp.where(z == m, col, 2 * NUM_OUTPUT), axis=1,
                      keepdims=True)
        onehot = (col == idx).astype(jnp.float32)
        note = onehot[:, :NUM_OUTPUT]
        ban = onehot[:, NUM_OUTPUT:]
        sample_p = sample_p + note * p[:, :NUM_OUTPUT]
        ban_p = ban_p + ban * p[:, NUM_OUTPUT:]
        sampled = jnp.minimum(sampled + note, 1.0)
        banned = jnp.minimum(banned + ban, 1.0)
    sampled_ref[...] = sampled
    probs_ref[...] = jnp.concatenate([sample_p, ban_p], axis=1)


def kernel(x, W_ih, W_hh, b_ih, b_hh, W_lin, b_lin):
    del W_hh
    B = x.shape[1]
    xf = x[0]
    Wp = jnp.zeros((3 * GP, AG + 2 * NUM_OUTPUT), jnp.float32)
    bias = b_ih + b_hh
    bp = jnp.zeros((3 * GP,), jnp.float32)
    for slot, (lo, hi) in enumerate(((0, H), (2 * H, 3 * H), (3 * H, 4 * H))):
        Wp = Wp.at[slot * GP:slot * GP + H].set(W_ih[lo:hi])
        bp = bp.at[slot * GP:slot * GP + H].set(bias[lo:hi])
    wx = Wp[:, :AG].T
    wsb = Wp[:, AG:].T
    wlin = W_lin.T
    skey = jax.random.key(42)
    G = jnp.stack([
        jax.random.gumbel(jax.random.fold_in(skey, v), (B, 2 * NUM_OUTPUT),
                          jnp.float32)
        for v in range(VOICES)
    ])

    grid = (B // B_BLK,)
    sampled, probs = pl.pallas_call(
        _poly_body,
        grid=grid,
        in_specs=[
            pl.BlockSpec((B_BLK, AG), lambda i: (i, 0)),
            pl.BlockSpec((AG, 3 * GP), lambda i: (0, 0)),
            pl.BlockSpec((2 * NUM_OUTPUT, 3 * GP), lambda i: (0, 0)),
            pl.BlockSpec((1, 3 * GP), lambda i: (0, 0)),
            pl.BlockSpec((H, 2 * NUM_OUTPUT), lambda i: (0, 0)),
            pl.BlockSpec((1, 2 * NUM_OUTPUT), lambda i: (0, 0)),
            pl.BlockSpec((VOICES, B_BLK, 2 * NUM_OUTPUT), lambda i: (0, i, 0)),
        ],
        out_specs=[
            pl.BlockSpec((B_BLK, NUM_OUTPUT), lambda i: (i, 0)),
            pl.BlockSpec((B_BLK, 2 * NUM_OUTPUT), lambda i: (i, 0)),
        ],
        out_shape=[
            jax.ShapeDtypeStruct((B, NUM_OUTPUT), jnp.float32),
            jax.ShapeDtypeStruct((B, 2 * NUM_OUTPUT), jnp.float32),
        ],
    )(xf, wx, wsb, bp[None, :], wlin, b_lin[None, :], G)
    return (sampled[None], probs[None])
